# baseline (speedup 1.0000x reference)
import functools

import jax
import jax.numpy as jnp
from jax import lax
from jax.experimental import pallas as pl
from jax.experimental.pallas import tpu as pltpu
from jax.experimental.pallas import tpu_sc as plsc

NC = 2
NS = 16
NW = NC * NS
L = 16


def _cdiv(a, b):
    return (a + b - 1) // b


def kernel(x, edge_index, relations, injection_node, node_batch,
           injection_node_batch, edge_attr):
    n, d = x.shape
    e = edge_index.shape[1]
    r, ed = relations.shape
    b = injection_node.shape[0]
    idt = edge_index.dtype
    en = e + n

    cx = _cdiv(n, NW)
    cx += (-cx) % 8
    cb = b // NW
    ce = e // NW

    mesh = plsc.VectorSubcoreMesh(core_axis_name="c", subcore_axis_name="s",
                                  num_cores=NC, num_subcores=NS)

    sc_out_type = (
        jax.ShapeDtypeStruct((n,), idt),
        jax.ShapeDtypeStruct((n + b,), jnp.int32),
        jax.ShapeDtypeStruct((en,), jnp.int32),
        jax.ShapeDtypeStruct(((r + 1) * ed,), relations.dtype),
        jax.ShapeDtypeStruct((r + 1,), jnp.int32),
    )

    @functools.partial(
        pl.kernel,
        out_type=sc_out_type,
        mesh=mesh,
        compiler_params=pltpu.CompilerParams(needs_layout_passes=False),
        scratch_types=[
            pltpu.VMEM((cx,), jnp.int32),
            pltpu.VMEM((b,), jnp.int32),
            pltpu.VMEM((cx,), jnp.int32),
            pltpu.VMEM((ce,), jnp.int32),
            pltpu.VMEM((cx,), jnp.int32),
            pltpu.VMEM((r * ed,), jnp.float32),
            pltpu.VMEM((ed,), jnp.float32),
            pltpu.VMEM((2 * L,), jnp.int32),
            pltpu.SemaphoreType.DMA,
            pltpu.SemaphoreType.DMA,
        ],
    )
    def sc_injector(rel_h, nb_h, inb_h, ea_h,
                    src_h, xinj_h, einj_h, relo_h, rinj_h,
                    nb_v, tbl_v, src_v, zero_v, one_v, relb_v, eab_v, rv_v,
                    sem_s, sem_out):
        wid = lax.axis_index("s") * NC + lax.axis_index("c")
        s = jnp.minimum(wid * cx, n - cx)
        ec = wid * ce

        h_small = [
            pltpu.async_copy(nb_h.at[pl.ds(s, cx)], nb_v, sem_s),
            pltpu.async_copy(inb_h, tbl_v, sem_s),
        ]

        def zofill(j, _):
            zero_v[pl.ds(j * L, L)] = jnp.zeros((L,), jnp.int32)
            return 0
        lax.fori_loop(0, ce // L, zofill, 0)

        def onefill(j, _):
            one_v[pl.ds(j * L, L)] = jnp.ones((L,), jnp.int32)
            return 0
        lax.fori_loop(0, cx // L, onefill, 0)

        for h in h_small:
            h.wait()

        def srcfill(j, _):
            o = j * L
            idx = nb_v[pl.ds(o, L)]
            src_v[pl.ds(o, L)] = plsc.load_gather(tbl_v, [idx]) + n
            return 0
        lax.fori_loop(0, cx // L, srcfill, 0)

        h_out = [
            pltpu.async_copy(src_v, src_h.at[pl.ds(s, cx)], sem_out),
            pltpu.async_copy(zero_v.at[pl.ds(0, cx)],
                             xinj_h.at[pl.ds(s, cx)], sem_out),
            pltpu.async_copy(one_v.at[pl.ds(0, cb)],
                             xinj_h.at[pl.ds(n + wid * cb, cb)], sem_out),
            pltpu.async_copy(zero_v, einj_h.at[pl.ds(ec, ce)], sem_out),
            pltpu.async_copy(one_v, einj_h.at[pl.ds(e + s, cx)], sem_out),
        ]

        @pl.when(wid == 0)
        def _():
            pltpu.sync_copy(rel_h, relb_v)
            pltpu.sync_copy(ea_h, eab_v)
            rv_v[pl.ds(0, L)] = jnp.zeros((L,), jnp.int32)
            rv_v[pl.ds(L, L)] = jnp.where(lax.iota(jnp.int32, L) == 0, 1, 0)
            pltpu.sync_copy(relb_v, relo_h.at[pl.ds(0, r * ed)])
            pltpu.sync_copy(eab_v, relo_h.at[pl.ds(r * ed, ed)])
            pltpu.sync_copy(rv_v.at[pl.ds(0, r + 1)], rinj_h)

        for h in h_out:
            h.wait()

    src_row, x_inj, e_inj, rel1d, r_inj = sc_injector(
        relations.reshape(-1), node_batch, injection_node_batch, edge_attr)

    CR = 2000
    nxb = n // CR

    def xcat_body(x_ref, inj_ref, o_ref):
        j = pl.program_id(0)
        pad = jnp.zeros((CR - b, d), x_ref.dtype)
        inj_full = jnp.concatenate([inj_ref[...], pad], axis=0)
        o_ref[...] = jnp.where(j < nxb, x_ref[...], inj_full)

    x_out = pl.pallas_call(
        xcat_body,
        grid=(_cdiv(n + b, CR),),
        in_specs=[
            pl.BlockSpec((CR, d), lambda i: (jnp.minimum(i, nxb - 1), 0)),
            pl.BlockSpec((b, d), lambda i: (0, 0)),
        ],
        out_specs=pl.BlockSpec((CR, d), lambda i: (i, 0)),
        out_shape=jax.ShapeDtypeStruct((n + b, d), x.dtype),
        compiler_params=pltpu.CompilerParams(
            dimension_semantics=("arbitrary",)),
    )(x, injection_node)

    CT = 2048
    neb = e // CT
    lead = e - neb * CT
    npb = _cdiv(lead + n, CT)
    src_pad = jnp.concatenate([
        jnp.zeros((lead,), idt), src_row,
        jnp.zeros((npb * CT - lead - n,), idt)])
    src3 = src_pad.reshape(npb, 1, CT)

    def ecat_body(ei_ref, src_ref, o_ref):
        j = pl.program_id(0)
        colg = jax.lax.broadcasted_iota(idt, (3, CT), 1) + j * CT
        rowi = jax.lax.broadcasted_iota(idt, (3, CT), 0)
        srow = jnp.broadcast_to(src_ref[0], (3, CT))
        gen = jnp.where(rowi == 0, srow,
                        jnp.where(rowi == 1, jnp.full((3, CT), r, idt),
                                  colg - e))
        o_ref[...] = jnp.where(colg < e, ei_ref[...], gen)

    eio = pl.pallas_call(
        ecat_body,
        grid=(_cdiv(en, CT),),
        in_specs=[
            pl.BlockSpec((3, CT),
                         lambda i: (0, jnp.minimum(i, _cdiv(e, CT) - 1))),
            pl.BlockSpec((1, 1, CT),
                         lambda i: (jnp.clip(i - neb, 0, npb - 1), 0, 0)),
        ],
        out_specs=pl.BlockSpec((3, CT), lambda i: (0, i)),
        out_shape=jax.ShapeDtypeStruct((3, en), idt),
        compiler_params=pltpu.CompilerParams(
            dimension_semantics=("arbitrary",)),
    )(edge_index, src3)

    return (x_out, eio, rel1d.reshape(r + 1, ed), x_inj, e_inj, r_inj)


# speedup vs baseline: 2.5255x; 2.5255x over previous
import functools

import jax
import jax.numpy as jnp
from jax import lax
from jax.experimental import pallas as pl
from jax.experimental.pallas import tpu as pltpu
from jax.experimental.pallas import tpu_sc as plsc

NC = 2
NS = 16
NW = NC * NS
L = 16


def _cdiv(a, b):
    return (a + b - 1) // b


def kernel(x, edge_index, relations, injection_node, node_batch,
           injection_node_batch, edge_attr):
    n, d = x.shape
    e = edge_index.shape[1]
    r, ed = relations.shape
    b = injection_node.shape[0]
    idt = edge_index.dtype
    en = e + n

    cx = _cdiv(n, NW)
    cx += (-cx) % 8
    cb = b // NW
    ce = e // NW

    mesh = plsc.VectorSubcoreMesh(core_axis_name="c", subcore_axis_name="s",
                                  num_cores=NC, num_subcores=NS)

    sc_out_type = (
        jax.ShapeDtypeStruct((n,), idt),
        jax.ShapeDtypeStruct((n + b,), jnp.int32),
        jax.ShapeDtypeStruct((en,), jnp.int32),
        jax.ShapeDtypeStruct(((r + 1) * ed,), relations.dtype),
        jax.ShapeDtypeStruct((r + 1,), jnp.int32),
    )

    @functools.partial(
        pl.kernel,
        out_type=sc_out_type,
        mesh=mesh,
        compiler_params=pltpu.CompilerParams(needs_layout_passes=False),
        scratch_types=[
            pltpu.VMEM((cx,), jnp.int32),
            pltpu.VMEM((b,), jnp.int32),
            pltpu.VMEM((cx,), jnp.int32),
            pltpu.VMEM((ce,), jnp.int32),
            pltpu.VMEM((cx,), jnp.int32),
            pltpu.VMEM((r * ed,), jnp.float32),
            pltpu.VMEM((ed,), jnp.float32),
            pltpu.VMEM((2 * L,), jnp.int32),
            pltpu.SemaphoreType.DMA,
            pltpu.SemaphoreType.DMA,
        ],
    )
    def sc_injector(rel_h, nb_h, inb_h, ea_h,
                    src_h, xinj_h, einj_h, relo_h, rinj_h,
                    nb_v, tbl_v, src_v, zero_v, one_v, relb_v, eab_v, rv_v,
                    sem_s, sem_out):
        wid = lax.axis_index("s") * NC + lax.axis_index("c")
        s = jnp.minimum(wid * cx, n - cx)
        ec = wid * ce

        h_small = [
            pltpu.async_copy(nb_h.at[pl.ds(s, cx)], nb_v, sem_s),
            pltpu.async_copy(inb_h, tbl_v, sem_s),
        ]

        def zofill(j, _):
            zero_v[pl.ds(j * L, L)] = jnp.zeros((L,), jnp.int32)
            return 0
        lax.fori_loop(0, ce // L, zofill, 0)

        def onefill(j, _):
            one_v[pl.ds(j * L, L)] = jnp.ones((L,), jnp.int32)
            return 0
        lax.fori_loop(0, cx // L, onefill, 0)

        for h in h_small:
            h.wait()

        def srcfill(j, _):
            o = j * L
            idx = nb_v[pl.ds(o, L)]
            src_v[pl.ds(o, L)] = plsc.load_gather(tbl_v, [idx]) + n
            return 0
        lax.fori_loop(0, cx // L, srcfill, 0)

        h_out = [
            pltpu.async_copy(src_v, src_h.at[pl.ds(s, cx)], sem_out),
            pltpu.async_copy(zero_v.at[pl.ds(0, cx)],
                             xinj_h.at[pl.ds(s, cx)], sem_out),
            pltpu.async_copy(one_v.at[pl.ds(0, cb)],
                             xinj_h.at[pl.ds(n + wid * cb, cb)], sem_out),
            pltpu.async_copy(zero_v, einj_h.at[pl.ds(ec, ce)], sem_out),
            pltpu.async_copy(one_v, einj_h.at[pl.ds(e + s, cx)], sem_out),
        ]

        @pl.when(wid == 0)
        def _():
            pltpu.sync_copy(rel_h, relb_v)
            pltpu.sync_copy(ea_h, eab_v)
            rv_v[pl.ds(0, L)] = jnp.zeros((L,), jnp.int32)
            rv_v[pl.ds(L, L)] = jnp.where(lax.iota(jnp.int32, L) == 0, 1, 0)
            pltpu.sync_copy(relb_v, relo_h.at[pl.ds(0, r * ed)])
            pltpu.sync_copy(eab_v, relo_h.at[pl.ds(r * ed, ed)])
            pltpu.sync_copy(rv_v.at[pl.ds(0, r + 1)], rinj_h)

        for h in h_out:
            h.wait()

    src_row, x_inj, e_inj, rel1d, r_inj = sc_injector(
        relations.reshape(-1), node_batch, injection_node_batch, edge_attr)

    CR = 5000
    nxb = n // CR

    def xcat_body(x_ref, inj_ref, o_ref):
        j = pl.program_id(0)
        pad = jnp.zeros((CR - b, d), x_ref.dtype)
        inj_full = jnp.concatenate([inj_ref[...], pad], axis=0)
        o_ref[...] = jnp.where(j < nxb, x_ref[...], inj_full)

    x_out = pl.pallas_call(
        xcat_body,
        grid=(_cdiv(n + b, CR),),
        in_specs=[
            pl.BlockSpec((CR, d), lambda i: (jnp.minimum(i, nxb - 1), 0)),
            pl.BlockSpec((b, d), lambda i: (0, 0)),
        ],
        out_specs=pl.BlockSpec((CR, d), lambda i: (i, 0)),
        out_shape=jax.ShapeDtypeStruct((n + b, d), x.dtype),
        compiler_params=pltpu.CompilerParams(
            dimension_semantics=("arbitrary",)),
    )(x, injection_node)

    CT = 16384
    neb = e // CT
    lead = e - neb * CT
    npb = _cdiv(lead + n, CT)
    src_pad = jnp.concatenate([
        jnp.zeros((lead,), idt), src_row,
        jnp.zeros((npb * CT - lead - n,), idt)])
    src3 = src_pad.reshape(npb, 1, CT)

    def ecat_body(ei_ref, src_ref, o_ref):
        j = pl.program_id(0)
        colg = jax.lax.broadcasted_iota(idt, (3, CT), 1) + j * CT
        rowi = jax.lax.broadcasted_iota(idt, (3, CT), 0)
        srow = jnp.broadcast_to(src_ref[0], (3, CT))
        gen = jnp.where(rowi == 0, srow,
                        jnp.where(rowi == 1, jnp.full((3, CT), r, idt),
                                  colg - e))
        o_ref[...] = jnp.where(colg < e, ei_ref[...], gen)

    eio = pl.pallas_call(
        ecat_body,
        grid=(_cdiv(en, CT),),
        in_specs=[
            pl.BlockSpec((3, CT),
                         lambda i: (0, jnp.minimum(i, _cdiv(e, CT) - 1))),
            pl.BlockSpec((1, 1, CT),
                         lambda i: (jnp.clip(i - neb, 0, npb - 1), 0, 0)),
        ],
        out_specs=pl.BlockSpec((3, CT), lambda i: (0, i)),
        out_shape=jax.ShapeDtypeStruct((3, en), idt),
        compiler_params=pltpu.CompilerParams(
            dimension_semantics=("arbitrary",)),
    )(edge_index, src3)

    return (x_out, eio, rel1d.reshape(r + 1, ed), x_inj, e_inj, r_inj)


# speedup vs baseline: 2.9451x; 1.1661x over previous
import functools

import jax
import jax.numpy as jnp
from jax import lax
from jax.experimental import pallas as pl
from jax.experimental.pallas import tpu as pltpu
from jax.experimental.pallas import tpu_sc as plsc

NC = 2
NS = 16
NW = NC * NS
L = 16


def _cdiv(a, b):
    return (a + b - 1) // b


def kernel(x, edge_index, relations, injection_node, node_batch,
           injection_node_batch, edge_attr):
    n, d = x.shape
    e = edge_index.shape[1]
    r, ed = relations.shape
    b = injection_node.shape[0]
    idt = edge_index.dtype
    en = e + n

    CT = 16384
    neb = e // CT
    lead = e - neb * CT
    npb = _cdiv(lead + n, CT)
    spad = npb * CT

    cx = _cdiv(n, NW)
    cx += (-cx) % 8

    mesh = plsc.VectorSubcoreMesh(core_axis_name="c", subcore_axis_name="s",
                                  num_cores=NC, num_subcores=NS)

    @functools.partial(
        pl.kernel,
        out_type=jax.ShapeDtypeStruct((spad,), idt),
        mesh=mesh,
        compiler_params=pltpu.CompilerParams(needs_layout_passes=False),
        scratch_types=[
            pltpu.VMEM((cx,), jnp.int32),
            pltpu.VMEM((b,), jnp.int32),
            pltpu.VMEM((cx,), jnp.int32),
            pltpu.SemaphoreType.DMA,
            pltpu.SemaphoreType.DMA,
        ],
    )
    def sc_gather(nb_h, inb_h, src_h, nb_v, tbl_v, src_v, sem_s, sem_out):
        wid = lax.axis_index("s") * NC + lax.axis_index("c")
        s = jnp.minimum(wid * cx, n - cx)
        h_in = [
            pltpu.async_copy(nb_h.at[pl.ds(s, cx)], nb_v, sem_s),
            pltpu.async_copy(inb_h, tbl_v, sem_s),
        ]
        for h in h_in:
            h.wait()

        def srcfill(j, _):
            o = j * L
            idx = nb_v[pl.ds(o, L)]
            src_v[pl.ds(o, L)] = plsc.load_gather(tbl_v, [idx]) + n
            return 0
        lax.fori_loop(0, cx // L, srcfill, 0)

        pltpu.async_copy(src_v, src_h.at[pl.ds(lead + s, cx)], sem_out).wait()

    src_pad = sc_gather(node_batch, injection_node_batch)

    CR = 5000
    nxb = n // CR
    CRI = 4096
    ea2 = edge_attr.reshape(1, ed)

    def xcat_body(x_ref, inj_ref, rel_ref, ea_ref,
                  o_ref, xinj_ref, relo_ref, rinj_ref):
        j = pl.program_id(0)
        pad = jnp.zeros((CR - b, d), x_ref.dtype)
        inj_full = jnp.concatenate([inj_ref[...], pad], axis=0)
        o_ref[...] = jnp.where(j < nxb, x_ref[...], inj_full)
        rowg = jax.lax.broadcasted_iota(jnp.int32, (CRI,), 0) + j * CRI
        xinj_ref[...] = jnp.where(rowg < n, 0, 1).astype(jnp.int32)
        relo_ref[...] = jnp.concatenate([rel_ref[...], ea_ref[...]], axis=0)
        ri = jax.lax.broadcasted_iota(jnp.int32, (r + 1,), 0)
        rinj_ref[...] = jnp.where(ri < r, 0, 1).astype(jnp.int32)

    x_out, x_inj, rel_out, r_inj = pl.pallas_call(
        xcat_body,
        grid=(_cdiv(n + b, CR),),
        in_specs=[
            pl.BlockSpec((CR, d), lambda i: (jnp.minimum(i, nxb - 1), 0)),
            pl.BlockSpec((b, d), lambda i: (0, 0)),
            pl.BlockSpec((r, ed), lambda i: (0, 0)),
            pl.BlockSpec((1, ed), lambda i: (0, 0)),
        ],
        out_specs=[
            pl.BlockSpec((CR, d), lambda i: (i, 0)),
            pl.BlockSpec((CRI,), lambda i: (i,)),
            pl.BlockSpec((r + 1, ed), lambda i: (0, 0)),
            pl.BlockSpec((r + 1,), lambda i: (0,)),
        ],
        out_shape=[
            jax.ShapeDtypeStruct((n + b, d), x.dtype),
            jax.ShapeDtypeStruct((n + b,), jnp.int32),
            jax.ShapeDtypeStruct((r + 1, ed), relations.dtype),
            jax.ShapeDtypeStruct((r + 1,), jnp.int32),
        ],
        compiler_params=pltpu.CompilerParams(
            dimension_semantics=("arbitrary",)),
    )(x, injection_node, relations, ea2)

    def ecat_body(ei_ref, src_ref, o_ref, einj_ref):
        j = pl.program_id(0)

        @pl.when(j < neb)
        def _():
            o_ref[...] = ei_ref[...]
            einj_ref[...] = jnp.zeros((CT,), jnp.int32)

        @pl.when(j >= neb)
        def _():
            colg = jax.lax.broadcasted_iota(idt, (3, CT), 1) + j * CT
            rowi = jax.lax.broadcasted_iota(idt, (3, CT), 0)
            srow = jnp.broadcast_to(src_ref[...].reshape(1, CT), (3, CT))
            gen = jnp.where(rowi == 0, srow,
                            jnp.where(rowi == 1, jnp.full((3, CT), r, idt),
                                      colg - e))
            o_ref[...] = jnp.where(colg < e, ei_ref[...], gen)
            colg1 = jax.lax.broadcasted_iota(jnp.int32, (CT,), 0) + j * CT
            einj_ref[...] = jnp.where(colg1 < e, 0, 1).astype(jnp.int32)

    eio, e_inj = pl.pallas_call(
        ecat_body,
        grid=(_cdiv(en, CT),),
        in_specs=[
            pl.BlockSpec((3, CT),
                         lambda i: (0, jnp.minimum(i, _cdiv(e, CT) - 1))),
            pl.BlockSpec((CT,), lambda i: (jnp.clip(i - neb, 0, npb - 1),)),
        ],
        out_specs=[
            pl.BlockSpec((3, CT), lambda i: (0, i)),
            pl.BlockSpec((CT,), lambda i: (i,)),
        ],
        out_shape=[
            jax.ShapeDtypeStruct((3, en), idt),
            jax.ShapeDtypeStruct((en,), jnp.int32),
        ],
        compiler_params=pltpu.CompilerParams(
            dimension_semantics=("arbitrary",)),
    )(edge_index, src_pad)

    return (x_out, eio, rel_out, x_inj, e_inj, r_inj)
